# baseline (speedup 1.0000x reference)
import jax
import jax.numpy as jnp
from jax import lax
from jax.experimental import pallas as pl
from jax.experimental.pallas import tpu as pltpu
from jax.experimental.pallas import tpu_sc as plsc

BATCH = 16384
MAX_D = 20
DIM = 32
PADW = 128

_B = BATCH * MAX_D
_NW = 32
_PER_W = _B // _NW
_BLK = 128
_NBLK = _PER_W // _BLK
_GRP = 5
_NGRP = _NBLK // _GRP


def _emb_body(idx_hbm, table_hbm, out_hbm, idx_v, rows_v, gsem, osem):
    cid = lax.axis_index("c")
    sid = lax.axis_index("s")
    wid = sid * 2 + cid
    base = wid * _PER_W

    pltpu.sync_copy(idx_hbm.at[pl.ds(base, _PER_W)], idx_v)

    def group(g, carry):
        copies = []
        for j in range(_GRP):
            copies.append(
                pltpu.async_copy(
                    table_hbm.at[idx_v.at[pl.ds((g * _GRP + j) * _BLK, _BLK)]],
                    rows_v.at[j],
                    gsem,
                )
            )
        for c in copies:
            c.wait()
        outs = []
        for j in range(_GRP):
            outs.append(
                pltpu.async_copy(
                    rows_v.at[j, :, pl.ds(0, DIM)],
                    out_hbm.at[pl.ds(base + (g * _GRP + j) * _BLK, _BLK)],
                    osem,
                )
            )
        for c in outs:
            c.wait()
        return carry

    lax.fori_loop(0, _NGRP, group, 0)


@jax.jit
def _emb_call(x_flat, table_pad):
    mesh = plsc.VectorSubcoreMesh(core_axis_name="c", subcore_axis_name="s")
    f = pl.kernel(
        _emb_body,
        out_type=jax.ShapeDtypeStruct((_B, DIM), jnp.float32),
        mesh=mesh,
        scratch_types=[
            pltpu.VMEM((_PER_W,), jnp.int32),
            pltpu.VMEM((_GRP, _BLK, PADW), jnp.float32),
            pltpu.SemaphoreType.DMA,
            pltpu.SemaphoreType.DMA,
        ],
        compiler_params=pltpu.CompilerParams(use_tc_tiling_on_sc=False),
    )
    return f(x_flat, table_pad)


def kernel(x, domain_emb_weight):
    x_flat = x.reshape(_B).astype(jnp.int32)
    table_pad = jnp.pad(domain_emb_weight, ((0, 0), (0, PADW - DIM)))
    out = _emb_call(x_flat, table_pad)
    return out.reshape(BATCH, MAX_D, DIM)


# speedup vs baseline: 1.1609x; 1.1609x over previous
import jax
import jax.numpy as jnp
from jax import lax
from jax.experimental import pallas as pl
from jax.experimental.pallas import tpu as pltpu
from jax.experimental.pallas import tpu_sc as plsc

BATCH = 16384
MAX_D = 20
DIM = 32

_B = BATCH * MAX_D
_NW = 32
_PER_W = _B // _NW
_CH = 512
_NCH = _PER_W // _CH


def _emb_body(idx_hbm, table_hbm, out_hbm, idx_v, rows_v, gsem, osem):
    cid = lax.axis_index("c")
    sid = lax.axis_index("s")
    wid = sid * 2 + cid
    base = wid * _PER_W

    pltpu.sync_copy(idx_hbm.at[pl.ds(base, _PER_W)], idx_v)

    def chunk(c, carry):
        def enq(k16, carry2):
            v = idx_v[pl.ds(c * _CH + k16 * 16, 16)]
            for t in range(16):
                pltpu.async_copy(
                    table_hbm.at[pl.ds(v[t], 1)],
                    rows_v.at[pl.ds(k16 * 16 + t, 1)],
                    gsem,
                )
            return carry2

        lax.fori_loop(0, _CH // 16, enq, 0)

        def drain(k, carry2):
            pltpu.make_async_copy(
                table_hbm.at[pl.ds(0, 1)], rows_v.at[pl.ds(0, 1)], gsem
            ).wait()
            return carry2

        lax.fori_loop(0, _CH, drain, 0)

        pltpu.async_copy(
            rows_v, out_hbm.at[pl.ds(base + c * _CH, _CH)], osem
        ).wait()
        return carry

    lax.fori_loop(0, _NCH, chunk, 0)


@jax.jit
def _emb_call(x_flat, table):
    mesh = plsc.VectorSubcoreMesh(core_axis_name="c", subcore_axis_name="s")
    f = pl.kernel(
        _emb_body,
        out_type=jax.ShapeDtypeStruct((_B, DIM), jnp.float32),
        mesh=mesh,
        scratch_types=[
            pltpu.VMEM((_PER_W,), jnp.int32),
            pltpu.VMEM((_CH, DIM), jnp.float32),
            pltpu.SemaphoreType.DMA,
            pltpu.SemaphoreType.DMA,
        ],
        compiler_params=pltpu.CompilerParams(use_tc_tiling_on_sc=True),
    )
    return f(x_flat, table)


def kernel(x, domain_emb_weight):
    x_flat = x.reshape(_B).astype(jnp.int32)
    out = _emb_call(x_flat, domain_emb_weight)
    return out.reshape(BATCH, MAX_D, DIM)


# speedup vs baseline: 1.3820x; 1.1904x over previous
import jax
import jax.numpy as jnp
from jax import lax
from jax.experimental import pallas as pl
from jax.experimental.pallas import tpu as pltpu
from jax.experimental.pallas import tpu_sc as plsc

BATCH = 16384
MAX_D = 20
DIM = 32

_B = BATCH * MAX_D
_NW = 32
_PER_W = _B // _NW
_BPW = BATCH // _NW
_CHB = 16
_CH = _CHB * MAX_D
_NCH = _BPW // _CHB


def _emb_body(idx_hbm, table_hbm, out_hbm, idx_v, rows_v, gsem, osem):
    cid = lax.axis_index("c")
    sid = lax.axis_index("s")
    wid = sid * 2 + cid
    base = wid * _PER_W
    bbase = wid * _BPW

    pltpu.sync_copy(idx_hbm.at[pl.ds(base, _PER_W)], idx_v)

    def chunk(c, carry):
        def enq(k16, carry2):
            v = idx_v[pl.ds(c * _CH + k16 * 16, 16)]
            for t in range(16):
                pltpu.async_copy(
                    table_hbm.at[pl.ds(v[t], 1)],
                    rows_v.at[pl.ds(k16 * 16 + t, 1)],
                    gsem,
                )
            return carry2

        lax.fori_loop(0, _CH // 16, enq, 0)

        def drain(k, carry2):
            pltpu.make_async_copy(
                table_hbm.at[pl.ds(0, 1)], rows_v.at[pl.ds(0, 1)], gsem
            ).wait()
            return carry2

        lax.fori_loop(0, _CH, drain, 0)

        outs = []
        for q in range(_CHB):
            outs.append(
                pltpu.async_copy(
                    rows_v.at[pl.ds(q * MAX_D, MAX_D)],
                    out_hbm.at[bbase + c * _CHB + q],
                    osem,
                )
            )
        for o in outs:
            o.wait()
        return carry

    lax.fori_loop(0, _NCH, chunk, 0)


@jax.jit
def _emb_call(x_flat, table):
    mesh = plsc.VectorSubcoreMesh(core_axis_name="c", subcore_axis_name="s")
    f = pl.kernel(
        _emb_body,
        out_type=jax.ShapeDtypeStruct((BATCH, MAX_D, DIM), jnp.float32),
        mesh=mesh,
        scratch_types=[
            pltpu.VMEM((_PER_W,), jnp.int32),
            pltpu.VMEM((_CH, DIM), jnp.float32),
            pltpu.SemaphoreType.DMA,
            pltpu.SemaphoreType.DMA,
        ],
        compiler_params=pltpu.CompilerParams(use_tc_tiling_on_sc=True),
    )
    return f(x_flat, table)


def kernel(x, domain_emb_weight):
    x_flat = x.reshape(_B).astype(jnp.int32)
    return _emb_call(x_flat, domain_emb_weight)


# speedup vs baseline: 1.3991x; 1.0123x over previous
import jax
import jax.numpy as jnp
from jax import lax
from jax.experimental import pallas as pl
from jax.experimental.pallas import tpu as pltpu
from jax.experimental.pallas import tpu_sc as plsc

BATCH = 16384
MAX_D = 20
DIM = 32

_B = BATCH * MAX_D
_NW = 32
_PER_W = _B // _NW
_BPW = BATCH // _NW
_CHB = 16
_CH = _CHB * MAX_D
_NCH = _BPW // _CHB


def _emb_body(idx_hbm, table_hbm, out_hbm, idx_v, rows_v, gsem, osem):
    cid = lax.axis_index("c")
    sid = lax.axis_index("s")
    wid = sid * 2 + cid
    base = wid * _PER_W
    bbase = wid * _BPW

    pltpu.sync_copy(idx_hbm.at[pl.ds(base, _PER_W)], idx_v)

    def chunk(c, carry):
        def enq(k16, carry2):
            v = idx_v[pl.ds(c * _CH + k16 * 16, 16)]
            for t in range(16):
                pltpu.async_copy(
                    table_hbm.at[pl.ds(v[t], 1)],
                    rows_v.at[pl.ds(k16 * 16 + t, 1)],
                    gsem,
                )
            return carry2

        lax.fori_loop(0, _CH // 16, enq, 0)

        pltpu.make_async_copy(
            table_hbm.at[pl.ds(0, _CH)], rows_v, gsem
        ).wait()

        outs = []
        for q in range(_CHB):
            outs.append(
                pltpu.async_copy(
                    rows_v.at[pl.ds(q * MAX_D, MAX_D)],
                    out_hbm.at[bbase + c * _CHB + q],
                    osem,
                )
            )
        for o in outs:
            o.wait()
        return carry

    lax.fori_loop(0, _NCH, chunk, 0)


@jax.jit
def _emb_call(x_flat, table):
    mesh = plsc.VectorSubcoreMesh(core_axis_name="c", subcore_axis_name="s")
    f = pl.kernel(
        _emb_body,
        out_type=jax.ShapeDtypeStruct((BATCH, MAX_D, DIM), jnp.float32),
        mesh=mesh,
        scratch_types=[
            pltpu.VMEM((_PER_W,), jnp.int32),
            pltpu.VMEM((_CH, DIM), jnp.float32),
            pltpu.SemaphoreType.DMA,
            pltpu.SemaphoreType.DMA,
        ],
        compiler_params=pltpu.CompilerParams(use_tc_tiling_on_sc=True),
    )
    return f(x_flat, table)


def kernel(x, domain_emb_weight):
    x_flat = x.reshape(_B).astype(jnp.int32)
    return _emb_call(x_flat, domain_emb_weight)


# speedup vs baseline: 1.4470x; 1.0343x over previous
import jax
import jax.numpy as jnp
from jax import lax
from jax.experimental import pallas as pl
from jax.experimental.pallas import tpu as pltpu
from jax.experimental.pallas import tpu_sc as plsc

BATCH = 16384
MAX_D = 20
DIM = 32

_B = BATCH * MAX_D
_NW = 32
_PER_W = _B // _NW
_BPW = BATCH // _NW
_CHB = 16
_CH = _CHB * MAX_D
_NCH = _BPW // _CHB


def _emb_body(idx_hbm, table_hbm, out_hbm, idx_v, rows_v, gsem, osem):
    cid = lax.axis_index("c")
    sid = lax.axis_index("s")
    wid = sid * 2 + cid
    base = wid * _PER_W
    bbase = wid * _BPW

    pltpu.sync_copy(idx_hbm.at[pl.ds(base, _PER_W)], idx_v)

    bufs = (rows_v.at[0], rows_v.at[1])

    def enq_gathers(c, buf):
        def enq(k16, carry2):
            v = idx_v[pl.ds(c * _CH + k16 * 16, 16)]
            for t in range(16):
                pltpu.async_copy(
                    table_hbm.at[pl.ds(v[t], 1)],
                    buf.at[pl.ds(k16 * 16 + t, 1)],
                    gsem,
                )
            return carry2

        lax.fori_loop(0, _CH // 16, enq, 0)

    def drain_gathers(buf):
        pltpu.make_async_copy(table_hbm.at[pl.ds(0, _CH)], buf, gsem).wait()

    def enq_outs(c, buf):
        for q in range(_CHB):
            pltpu.async_copy(
                buf.at[pl.ds(q * MAX_D, MAX_D)],
                out_hbm.at[bbase + c * _CHB + q],
                osem,
            )

    def wait_outs(c, buf):
        for q in range(_CHB):
            pltpu.make_async_copy(
                buf.at[pl.ds(q * MAX_D, MAX_D)],
                out_hbm.at[bbase + c * _CHB + q],
                osem,
            ).wait()

    def pair(i, carry):
        c0 = 2 * i
        c1 = c0 + 1

        @pl.when(i > 0)
        def _():
            wait_outs(c0 - 2, bufs[0])

        enq_gathers(c0, bufs[0])
        drain_gathers(bufs[0])
        enq_outs(c0, bufs[0])

        @pl.when(i > 0)
        def _():
            wait_outs(c0 - 1, bufs[1])

        enq_gathers(c1, bufs[1])
        drain_gathers(bufs[1])
        enq_outs(c1, bufs[1])
        return carry

    lax.fori_loop(0, _NCH // 2, pair, 0)
    wait_outs(_NCH - 2, bufs[0])
    wait_outs(_NCH - 1, bufs[1])


@jax.jit
def _emb_call(x_flat, table):
    mesh = plsc.VectorSubcoreMesh(core_axis_name="c", subcore_axis_name="s")
    f = pl.kernel(
        _emb_body,
        out_type=jax.ShapeDtypeStruct((BATCH, MAX_D, DIM), jnp.float32),
        mesh=mesh,
        scratch_types=[
            pltpu.VMEM((_PER_W,), jnp.int32),
            pltpu.VMEM((2, _CH, DIM), jnp.float32),
            pltpu.SemaphoreType.DMA,
            pltpu.SemaphoreType.DMA,
        ],
        compiler_params=pltpu.CompilerParams(use_tc_tiling_on_sc=True),
    )
    return f(x_flat, table)


def kernel(x, domain_emb_weight):
    x_flat = x.reshape(_B).astype(jnp.int32)
    return _emb_call(x_flat, domain_emb_weight)
